# baseline (speedup 1.0000x reference)
import functools

import jax
import jax.numpy as jnp
from jax import lax
from jax.experimental import pallas as pl
from jax.experimental.pallas import tpu as pltpu
from jax.experimental.pallas import tpu_sc as plsc

N = 10000
D = 128
STEPS = 4
E = 320000
NSEL = 2000

NC = 2
NS = 16
NW = NC * NS

N_PAD = 10240
B = 2048
GRID = N_PAD // B

CH = 128
NCH = 80
E_PAD = NW * NCH * CH
RPS = N_PAD // NS
RCH = 128
NRC = RPS // RCH

SEL_PAD = 2048
SPW = SEL_PAD // NW

@functools.cache
def _mesh():
    return plsc.VectorSubcoreMesh(core_axis_name="c", subcore_axis_name="s",
                                  num_cores=NC, num_subcores=NS)


def _segsum_body(src_hbm, dst_hbm, m_hbm, z_hbm, out_hbm,
                 src_v, dst_v, buf_b, agg_sh, sem):
    c = lax.axis_index("c")
    s = lax.axis_index("s")
    wid = c * NS + s
    pltpu.sync_copy(src_hbm.at[wid], src_v)
    pltpu.sync_copy(dst_hbm.at[wid], dst_v)
    row0 = s * RPS

    def zbody(j, carry):
        pltpu.sync_copy(z_hbm, agg_sh.at[pl.ds(row0 + j * RCH, RCH)])
        return carry

    lax.fori_loop(0, NRC, zbody, 0)
    plsc.subcore_barrier()

    def ebody(g, carry):
        pltpu.async_copy(m_hbm.at[src_v.at[g]], buf_b, sem).wait()
        pltpu.sync_copy(buf_b, agg_sh.at[dst_v.at[g]], add=True)
        return carry

    lax.fori_loop(0, NCH, ebody, 0)
    plsc.subcore_barrier()

    def obody(j, carry):
        r = row0 + j * RCH
        pltpu.sync_copy(agg_sh.at[pl.ds(r, RCH)], out_hbm.at[c, pl.ds(r, RCH)])
        return carry

    lax.fori_loop(0, NRC, obody, 0)


@functools.cache
def _segsum_call():
    return pl.kernel(
        _segsum_body,
        out_type=jax.ShapeDtypeStruct((NC, N_PAD, D), jnp.float32),
        mesh=_mesh(),
        scratch_types=[
            pltpu.VMEM((NCH, CH), jnp.int32),
            pltpu.VMEM((NCH, CH), jnp.int32),
            pltpu.VMEM((CH, D), jnp.float32),
            pltpu.VMEM_SHARED((N_PAD, D), jnp.float32),
            pltpu.SemaphoreType.DMA,
        ],
    )


def _gather_body(h_hbm, idx_hbm, out_hbm, idx_v, rows_v, sem):
    c = lax.axis_index("c")
    s = lax.axis_index("s")
    wid = c * NS + s
    pltpu.sync_copy(idx_hbm.at[wid], idx_v)
    pltpu.async_copy(h_hbm.at[idx_v], rows_v, sem).wait()
    pltpu.sync_copy(rows_v, out_hbm.at[pl.ds(wid * SPW, SPW)])


@functools.cache
def _gather_call():
    return pl.kernel(
        _gather_body,
        out_type=jax.ShapeDtypeStruct((SEL_PAD, D), jnp.float32),
        mesh=_mesh(),
        scratch_types=[
            pltpu.VMEM((SPW,), jnp.int32),
            pltpu.VMEM((SPW, D), jnp.float32),
            pltpu.SemaphoreType.DMA,
        ],
    )


def _init_body(x_ref, wr_ref, br_ref, wc_ref, h_ref, m_ref):
    h = jnp.dot(x_ref[...], wr_ref[...],
                preferred_element_type=jnp.float32) + br_ref[...]
    h_ref[...] = h
    m_ref[...] = jnp.dot(h, wc_ref[...], preferred_element_type=jnp.float32)


def _init_call(x, wr, br, wc):
    return pl.pallas_call(
        _init_body,
        grid=(GRID,),
        in_specs=[
            pl.BlockSpec((B, D), lambda i: (i, 0)),
            pl.BlockSpec((D, D), lambda i: (0, 0)),
            pl.BlockSpec((1, D), lambda i: (0, 0)),
            pl.BlockSpec((D, D), lambda i: (0, 0)),
        ],
        out_specs=[
            pl.BlockSpec((B, D), lambda i: (i, 0)),
            pl.BlockSpec((B, D), lambda i: (i, 0)),
        ],
        out_shape=[
            jax.ShapeDtypeStruct((N_PAD, D), jnp.float32),
            jax.ShapeDtypeStruct((N_PAD, D), jnp.float32),
        ],
    )(x, wr, br, wc)


def _gru_body(agg_ref, h_ref, wi_ref, wh_ref, bi_ref, bh_ref, wc_ref,
              hn_ref, mn_ref):
    agg = agg_ref[0] + agg_ref[1]
    h = h_ref[...]
    gi = jnp.dot(agg, wi_ref[...],
                 preferred_element_type=jnp.float32) + bi_ref[...]
    gh = jnp.dot(h, wh_ref[...],
                 preferred_element_type=jnp.float32) + bh_ref[...]
    r = jax.nn.sigmoid(gi[:, :D] + gh[:, :D])
    z = jax.nn.sigmoid(gi[:, D:2 * D] + gh[:, D:2 * D])
    n = jnp.tanh(gi[:, 2 * D:] + r * gh[:, 2 * D:])
    hn = (1.0 - z) * n + z * h
    hn_ref[...] = hn
    mn_ref[...] = jnp.dot(hn, wc_ref[...], preferred_element_type=jnp.float32)


def _gru_call(aggs, h, wi, wh, bi, bh, wc):
    return pl.pallas_call(
        _gru_body,
        grid=(GRID,),
        in_specs=[
            pl.BlockSpec((NC, B, D), lambda i: (0, i, 0)),
            pl.BlockSpec((B, D), lambda i: (i, 0)),
            pl.BlockSpec((D, 3 * D), lambda i: (0, 0)),
            pl.BlockSpec((D, 3 * D), lambda i: (0, 0)),
            pl.BlockSpec((1, 3 * D), lambda i: (0, 0)),
            pl.BlockSpec((1, 3 * D), lambda i: (0, 0)),
            pl.BlockSpec((D, D), lambda i: (0, 0)),
        ],
        out_specs=[
            pl.BlockSpec((B, D), lambda i: (i, 0)),
            pl.BlockSpec((B, D), lambda i: (i, 0)),
        ],
        out_shape=[
            jax.ShapeDtypeStruct((N_PAD, D), jnp.float32),
            jax.ShapeDtypeStruct((N_PAD, D), jnp.float32),
        ],
    )(aggs, h, wi, wh, bi, bh, wc)


def _tail_body(sel_ref, wl_ref, bl_ref, o_ref):
    o_ref[...] = jax.nn.sigmoid(
        jnp.dot(sel_ref[...], wl_ref[...],
                preferred_element_type=jnp.float32) + bl_ref[...])


def _tail_call(sel, wl, bl):
    return pl.pallas_call(
        _tail_body,
        out_shape=jax.ShapeDtypeStruct((SEL_PAD, 1), jnp.float32),
    )(sel, wl, bl)


def kernel(x, edge_index, idx, W_red, b_red, W_conv, W_ih, W_hh,
           b_ih, b_hh, W_lin, b_lin):
    f32 = jnp.float32
    x_p = jnp.zeros((N_PAD, D), f32).at[:N].set(x.astype(f32))
    src = edge_index[0].astype(jnp.int32)
    dst = edge_index[1].astype(jnp.int32)
    src_p = jnp.zeros((E_PAD,), jnp.int32).at[:E].set(src).reshape(NW, NCH, CH)
    dst_p = jnp.full((E_PAD,), N, jnp.int32).at[:E].set(dst).reshape(NW, NCH, CH)
    idx_p = (jnp.zeros((SEL_PAD,), jnp.int32)
             .at[:NSEL].set(idx.astype(jnp.int32)).reshape(NW, SPW))
    zeros_blk = jnp.zeros((RCH, D), f32)

    wr = W_red.T.astype(f32)
    br = b_red.reshape(1, D).astype(f32)
    wi = W_ih.T.astype(f32)
    wh = W_hh.T.astype(f32)
    bi = b_ih.reshape(1, 3 * D).astype(f32)
    bh = b_hh.reshape(1, 3 * D).astype(f32)

    h, m = _init_call(x_p, wr, br, W_conv[0].astype(f32))
    for i in range(STEPS):
        aggs = _segsum_call()(src_p, dst_p, m, zeros_blk)
        h, m = _gru_call(aggs, h, wi, wh, bi, bh,
                         W_conv[(i + 1) % STEPS].astype(f32))
    sel = _gather_call()(h, idx_p)
    out = _tail_call(sel, W_lin.T.astype(f32), b_lin.reshape(1, 1).astype(f32))
    return out[:NSEL]


# speedup vs baseline: 1.0113x; 1.0113x over previous
import functools

import jax
import jax.numpy as jnp
from jax import lax
from jax.experimental import pallas as pl
from jax.experimental.pallas import tpu as pltpu
from jax.experimental.pallas import tpu_sc as plsc

N = 10000
D = 128
STEPS = 4
E = 320000
NSEL = 2000

NC = 2
NS = 16
NW = NC * NS

N_PAD = 10240
B = 2048
GRID = N_PAD // B

CH = 128
NCH = 80
E_PAD = NW * NCH * CH
RPS = N_PAD // NS
RCH = 128
NRC = RPS // RCH

SEL_PAD = 2048
SPW = SEL_PAD // NW

@functools.cache
def _mesh():
    return plsc.VectorSubcoreMesh(core_axis_name="c", subcore_axis_name="s",
                                  num_cores=NC, num_subcores=NS)


def _segsum_body(src_hbm, dst_hbm, m_hbm, z_hbm, out_hbm,
                 src_v, dst_v, buf_b, agg_sh, sem):
    c = lax.axis_index("c")
    s = lax.axis_index("s")
    wid = c * NS + s
    pltpu.sync_copy(src_hbm.at[wid], src_v)
    pltpu.sync_copy(dst_hbm.at[wid], dst_v)
    pltpu.sync_copy(z_hbm, buf_b)
    row0 = s * RPS

    def zbody(j, carry):
        pltpu.sync_copy(buf_b, agg_sh.at[pl.ds(row0 + j * RCH, RCH)])
        return carry

    lax.fori_loop(0, NRC, zbody, 0)
    plsc.subcore_barrier()

    def ebody(g, carry):
        pltpu.async_copy(m_hbm.at[src_v.at[g]], buf_b, sem).wait()
        pltpu.sync_copy(buf_b, agg_sh.at[dst_v.at[g]], add=True)
        return carry

    lax.fori_loop(0, NCH, ebody, 0)
    plsc.subcore_barrier()

    def obody(j, carry):
        r = row0 + j * RCH
        pltpu.sync_copy(agg_sh.at[pl.ds(r, RCH)], buf_b)
        pltpu.sync_copy(buf_b, out_hbm.at[c, pl.ds(r, RCH)])
        return carry

    lax.fori_loop(0, NRC, obody, 0)


@functools.cache
def _segsum_call():
    return pl.kernel(
        _segsum_body,
        out_type=jax.ShapeDtypeStruct((NC, N_PAD, D), jnp.float32),
        mesh=_mesh(),
        scratch_types=[
            pltpu.VMEM((NCH, CH), jnp.int32),
            pltpu.VMEM((NCH, CH), jnp.int32),
            pltpu.VMEM((CH, D), jnp.float32),
            pltpu.VMEM_SHARED((N_PAD, D), jnp.float32),
            pltpu.SemaphoreType.DMA,
        ],
    )


def _gather_body(h_hbm, idx_hbm, out_hbm, idx_v, rows_v, sem):
    c = lax.axis_index("c")
    s = lax.axis_index("s")
    wid = c * NS + s
    pltpu.sync_copy(idx_hbm.at[wid], idx_v)
    pltpu.async_copy(h_hbm.at[idx_v], rows_v, sem).wait()
    pltpu.sync_copy(rows_v, out_hbm.at[pl.ds(wid * SPW, SPW)])


@functools.cache
def _gather_call():
    return pl.kernel(
        _gather_body,
        out_type=jax.ShapeDtypeStruct((SEL_PAD, D), jnp.float32),
        mesh=_mesh(),
        scratch_types=[
            pltpu.VMEM((SPW,), jnp.int32),
            pltpu.VMEM((SPW, D), jnp.float32),
            pltpu.SemaphoreType.DMA,
        ],
    )


def _init_body(x_ref, wr_ref, br_ref, wc_ref, h_ref, m_ref):
    h = jnp.dot(x_ref[...], wr_ref[...],
                preferred_element_type=jnp.float32) + br_ref[...]
    h_ref[...] = h
    m_ref[...] = jnp.dot(h, wc_ref[...], preferred_element_type=jnp.float32)


def _init_call(x, wr, br, wc):
    return pl.pallas_call(
        _init_body,
        grid=(GRID,),
        in_specs=[
            pl.BlockSpec((B, D), lambda i: (i, 0)),
            pl.BlockSpec((D, D), lambda i: (0, 0)),
            pl.BlockSpec((1, D), lambda i: (0, 0)),
            pl.BlockSpec((D, D), lambda i: (0, 0)),
        ],
        out_specs=[
            pl.BlockSpec((B, D), lambda i: (i, 0)),
            pl.BlockSpec((B, D), lambda i: (i, 0)),
        ],
        out_shape=[
            jax.ShapeDtypeStruct((N_PAD, D), jnp.float32),
            jax.ShapeDtypeStruct((N_PAD, D), jnp.float32),
        ],
    )(x, wr, br, wc)


def _gru_body(agg_ref, h_ref, wi_ref, wh_ref, bi_ref, bh_ref, wc_ref,
              hn_ref, mn_ref):
    agg = agg_ref[0] + agg_ref[1]
    h = h_ref[...]
    gi = jnp.dot(agg, wi_ref[...],
                 preferred_element_type=jnp.float32) + bi_ref[...]
    gh = jnp.dot(h, wh_ref[...],
                 preferred_element_type=jnp.float32) + bh_ref[...]
    r = jax.nn.sigmoid(gi[:, :D] + gh[:, :D])
    z = jax.nn.sigmoid(gi[:, D:2 * D] + gh[:, D:2 * D])
    n = jnp.tanh(gi[:, 2 * D:] + r * gh[:, 2 * D:])
    hn = (1.0 - z) * n + z * h
    hn_ref[...] = hn
    mn_ref[...] = jnp.dot(hn, wc_ref[...], preferred_element_type=jnp.float32)


def _gru_call(aggs, h, wi, wh, bi, bh, wc):
    return pl.pallas_call(
        _gru_body,
        grid=(GRID,),
        in_specs=[
            pl.BlockSpec((NC, B, D), lambda i: (0, i, 0)),
            pl.BlockSpec((B, D), lambda i: (i, 0)),
            pl.BlockSpec((D, 3 * D), lambda i: (0, 0)),
            pl.BlockSpec((D, 3 * D), lambda i: (0, 0)),
            pl.BlockSpec((1, 3 * D), lambda i: (0, 0)),
            pl.BlockSpec((1, 3 * D), lambda i: (0, 0)),
            pl.BlockSpec((D, D), lambda i: (0, 0)),
        ],
        out_specs=[
            pl.BlockSpec((B, D), lambda i: (i, 0)),
            pl.BlockSpec((B, D), lambda i: (i, 0)),
        ],
        out_shape=[
            jax.ShapeDtypeStruct((N_PAD, D), jnp.float32),
            jax.ShapeDtypeStruct((N_PAD, D), jnp.float32),
        ],
    )(aggs, h, wi, wh, bi, bh, wc)


def _tail_body(sel_ref, wl_ref, bl_ref, o_ref):
    o_ref[...] = jax.nn.sigmoid(
        jnp.dot(sel_ref[...], wl_ref[...],
                preferred_element_type=jnp.float32) + bl_ref[...])


def _tail_call(sel, wl, bl):
    return pl.pallas_call(
        _tail_body,
        out_shape=jax.ShapeDtypeStruct((SEL_PAD, 1), jnp.float32),
    )(sel, wl, bl)


def kernel(x, edge_index, idx, W_red, b_red, W_conv, W_ih, W_hh,
           b_ih, b_hh, W_lin, b_lin):
    f32 = jnp.float32
    x_p = jnp.zeros((N_PAD, D), f32).at[:N].set(x.astype(f32))
    src = edge_index[0].astype(jnp.int32)
    dst = edge_index[1].astype(jnp.int32)
    src_p = jnp.zeros((E_PAD,), jnp.int32).at[:E].set(src).reshape(NW, NCH, CH)
    dst_p = jnp.full((E_PAD,), N, jnp.int32).at[:E].set(dst).reshape(NW, NCH, CH)
    idx_p = (jnp.zeros((SEL_PAD,), jnp.int32)
             .at[:NSEL].set(idx.astype(jnp.int32)).reshape(NW, SPW))
    zeros_blk = jnp.zeros((RCH, D), f32)

    wr = W_red.T.astype(f32)
    br = b_red.reshape(1, D).astype(f32)
    wi = W_ih.T.astype(f32)
    wh = W_hh.T.astype(f32)
    bi = b_ih.reshape(1, 3 * D).astype(f32)
    bh = b_hh.reshape(1, 3 * D).astype(f32)

    h, m = _init_call(x_p, wr, br, W_conv[0].astype(f32))
    for i in range(STEPS):
        aggs = _segsum_call()(src_p, dst_p, m, zeros_blk)
        h, m = _gru_call(aggs, h, wi, wh, bi, bh,
                         W_conv[(i + 1) % STEPS].astype(f32))
    sel = _gather_call()(h, idx_p)
    out = _tail_call(sel, W_lin.T.astype(f32), b_lin.reshape(1, 1).astype(f32))
    return out[:NSEL]
